# baseline (speedup 1.0000x reference)
import functools

import jax
import jax.numpy as jnp
from jax import lax
from jax.experimental import pallas as pl
from jax.experimental.pallas import tpu as pltpu
from jax.experimental.pallas import tpu_sc as plsc

_N = 10000
_NACC = 10016
_E = 320000
_NC = 2
_NS = 16
_CHUNK = 128
_NCHUNK = 80
_EPAD = _NC * _NS * _NCHUNK * _CHUNK
_ROWS_PER_TILE = 624

_mesh = plsc.VectorSubcoreMesh(core_axis_name="c", subcore_axis_name="s")
_sc_params = pltpu.CompilerParams(use_tc_tiling_on_sc=False)


@functools.partial(
    pl.kernel,
    out_type=jax.ShapeDtypeStruct((_NC, _N), jnp.float32),
    mesh=_mesh,
    compiler_params=_sc_params,
    scratch_types=[
        pltpu.VMEM((_NCHUNK, _CHUNK), jnp.int32),
        pltpu.VMEM((_CHUNK,), jnp.float32),
        pltpu.VMEM_SHARED((_NACC,), jnp.float32),
    ],
)
def _sc_degree(row_hbm, zeros_hbm, out_hbm, ridx_v, ones_v, deg_sp):
    c = lax.axis_index("c")
    s = lax.axis_index("s")

    @pl.when(s == 0)
    def _():
        pltpu.sync_copy(zeros_hbm, deg_sp)

    pltpu.sync_copy(row_hbm.at[c, s], ridx_v)
    for i in range(_CHUNK // 16):
        ones_v[pl.ds(i * 16, 16)] = jnp.ones((16,), jnp.float32)
    plsc.subcore_barrier()

    def body(j, carry):
        pltpu.sync_copy(ones_v, deg_sp.at[ridx_v.at[j]], add=True)
        return carry

    lax.fori_loop(0, _NCHUNK, body, 0)
    plsc.subcore_barrier()

    @pl.when(s == 0)
    def _():
        pltpu.sync_copy(deg_sp.at[pl.ds(0, _N)], out_hbm.at[c])


@functools.partial(
    pl.kernel,
    out_type=jax.ShapeDtypeStruct((_NC, _N, 16), jnp.float32),
    mesh=_mesh,
    compiler_params=_sc_params,
    scratch_types=[
        pltpu.VMEM((_NCHUNK, _CHUNK), jnp.int32),
        pltpu.VMEM((_NCHUNK, _CHUNK), jnp.int32),
        pltpu.VMEM((_CHUNK, 16), jnp.float32),
        pltpu.VMEM((_CHUNK, 16), jnp.float32),
        pltpu.VMEM_SHARED((_NACC, 16), jnp.float32),
        pltpu.SemaphoreType.DMA,
        pltpu.SemaphoreType.DMA,
    ],
)
def _sc_gather_scatter(row_hbm, col_hbm, table_hbm, zeros_hbm, out_hbm,
                       ridx_v, cidx_v, buf_a, buf_b, acc_sp, sem_a, sem_b):
    c = lax.axis_index("c")
    s = lax.axis_index("s")

    @pl.when(s == 0)
    def _():
        pltpu.sync_copy(zeros_hbm, acc_sp)

    pltpu.sync_copy(row_hbm.at[c, s], ridx_v)
    pltpu.sync_copy(col_hbm.at[c, s], cidx_v)
    plsc.subcore_barrier()

    def wait_gather(buf, sem):
        pltpu.make_async_copy(table_hbm.at[cidx_v.at[0]], buf, sem).wait()

    pltpu.async_copy(table_hbm.at[cidx_v.at[0]], buf_a, sem_a)

    def body(i, carry):
        j = 2 * i
        pltpu.async_copy(table_hbm.at[cidx_v.at[j + 1]], buf_b, sem_b)
        wait_gather(buf_a, sem_a)
        pltpu.sync_copy(buf_a, acc_sp.at[ridx_v.at[j]], add=True)

        @pl.when(j + 2 < _NCHUNK)
        def _():
            pltpu.async_copy(table_hbm.at[cidx_v.at[j + 2]], buf_a, sem_a)

        wait_gather(buf_b, sem_b)
        pltpu.sync_copy(buf_b, acc_sp.at[ridx_v.at[j + 1]], add=True)
        return carry

    lax.fori_loop(0, _NCHUNK // 2, body, 0)
    plsc.subcore_barrier()

    @pl.when(s < _NS - 1)
    def _():
        rows = pl.ds(s * _ROWS_PER_TILE, _ROWS_PER_TILE)
        pltpu.sync_copy(acc_sp.at[rows], out_hbm.at[c].at[rows])

    @pl.when(s == _NS - 1)
    def _():
        last = pl.ds((_NS - 1) * _ROWS_PER_TILE, _N - (_NS - 1) * _ROWS_PER_TILE)
        pltpu.sync_copy(acc_sp.at[last], out_hbm.at[c].at[last])


def _mm1_body(x_ref, w0_ref, w1_ref, z_ref, y_ref):
    xb = x_ref[...]
    z_ref[...] = jnp.dot(xb, w0_ref[...], preferred_element_type=jnp.float32)
    y_ref[...] = jnp.dot(xb, w1_ref[...], preferred_element_type=jnp.float32)


def _tc_mm1(x, w0, w1):
    blk = 1000
    return pl.pallas_call(
        _mm1_body,
        grid=(_N // blk,),
        in_specs=[
            pl.BlockSpec((blk, 128), lambda i: (i, 0)),
            pl.BlockSpec((128, 16), lambda i: (0, 0)),
            pl.BlockSpec((128, 16), lambda i: (0, 0)),
        ],
        out_specs=[
            pl.BlockSpec((blk, 16), lambda i: (i, 0)),
            pl.BlockSpec((blk, 16), lambda i: (i, 0)),
        ],
        out_shape=[
            jax.ShapeDtypeStruct((_N, 16), jnp.float32),
            jax.ShapeDtypeStruct((_N, 16), jnp.float32),
        ],
    )(x, w0, w1)


def _prep_body(degp_ref, y1_ref, dis_ref, y1p_ref):
    deg = degp_ref[:, 0:1] + degp_ref[:, 1:2]
    dis = jnp.where(deg > 0.0, lax.rsqrt(jnp.where(deg > 0.0, deg, 1.0)), 0.0)
    dis_ref[...] = dis
    y1p_ref[...] = dis * y1_ref[...]


def _tc_prep(degp_t, y1):
    return pl.pallas_call(
        _prep_body,
        out_shape=[
            jax.ShapeDtypeStruct((_N, 1), jnp.float32),
            jax.ShapeDtypeStruct((_N, 16), jnp.float32),
        ],
    )(degp_t, y1)


def _l1_body(z_ref, sp_ref, dis_ref, b_ref, h_ref, g_ref):
    ssum = sp_ref[0] + sp_ref[1]
    h = jnp.maximum(z_ref[...] - dis_ref[...] * ssum + b_ref[...], 0.0)
    h_ref[...] = h
    g_ref[...] = dis_ref[...] * h


def _tc_layer1(z1, s1p, dis, b1):
    return pl.pallas_call(
        _l1_body,
        out_shape=[
            jax.ShapeDtypeStruct((_N, 16), jnp.float32),
            jax.ShapeDtypeStruct((_N, 16), jnp.float32),
        ],
    )(z1, s1p, dis, b1)


def _l2_body(h_ref, sp_ref, dis_ref, w0_ref, w1_ref, b_ref, out_ref):
    t2 = -dis_ref[...] * (sp_ref[0] + sp_ref[1])
    logits = (
        jnp.dot(h_ref[...], w0_ref[...], preferred_element_type=jnp.float32)
        + jnp.dot(t2, w1_ref[...], preferred_element_type=jnp.float32)
        + b_ref[...]
    )
    m = jnp.max(logits, axis=1, keepdims=True)
    e = jnp.exp(logits - m)
    lse = jnp.log(jnp.sum(e, axis=1, keepdims=True))
    out_ref[...] = logits - m - lse


def _tc_layer2(h, s2p, dis, w0, w1, b2):
    return pl.pallas_call(
        _l2_body,
        out_shape=jax.ShapeDtypeStruct((_N, 64), jnp.float32),
    )(h, s2p, dis, w0, w1, b2)


def kernel(x, edge_index, W0_1, W1_1, b1, W0_2, W1_2, b2):
    npad = _EPAD - _E
    row = jnp.concatenate([edge_index[0], jnp.full((npad,), _N, jnp.int32)])
    col = jnp.concatenate([edge_index[1], jnp.zeros((npad,), jnp.int32)])
    row = row.reshape(_NC, _NS, _NCHUNK, _CHUNK)
    col = col.reshape(_NC, _NS, _NCHUNK, _CHUNK)
    zeros1 = jnp.zeros((_NACC,), jnp.float32)
    zeros2 = jnp.zeros((_NACC, 16), jnp.float32)

    degp = _sc_degree(row, zeros1)
    z1, y1 = _tc_mm1(x, W0_1, W1_1)
    dis, y1p = _tc_prep(degp.T, y1)
    s1p = _sc_gather_scatter(row, col, y1p, zeros2)
    h, g = _tc_layer1(z1, s1p, dis, b1)
    s2p = _sc_gather_scatter(row, col, g, zeros2)
    return _tc_layer2(h, s2p, dis, W0_2, W1_2, b2)


# speedup vs baseline: 1.2112x; 1.2112x over previous
import functools

import jax
import jax.numpy as jnp
from jax import lax
from jax.experimental import pallas as pl
from jax.experimental.pallas import tpu as pltpu
from jax.experimental.pallas import tpu_sc as plsc

_N = 10000
_NACC = 10016
_E = 320000
_NC = 2
_NS = 16
_CHUNK = 80
_NCHUNK = 125
_EPAD = _NC * _NS * _NCHUNK * _CHUNK
_ROWS_PER_TILE = 624

_mesh = plsc.VectorSubcoreMesh(core_axis_name="c", subcore_axis_name="s")
_sc_params = pltpu.CompilerParams(use_tc_tiling_on_sc=False)


@functools.partial(
    pl.kernel,
    out_type=jax.ShapeDtypeStruct((_NC, _N), jnp.float32),
    mesh=_mesh,
    compiler_params=_sc_params,
    scratch_types=[
        pltpu.VMEM((_NCHUNK, _CHUNK), jnp.int32),
        pltpu.VMEM((_CHUNK,), jnp.float32),
        pltpu.VMEM_SHARED((_NACC,), jnp.float32),
    ],
)
def _sc_degree(row_hbm, zeros_hbm, out_hbm, ridx_v, ones_v, deg_sp):
    c = lax.axis_index("c")
    s = lax.axis_index("s")

    @pl.when(s == 0)
    def _():
        pltpu.sync_copy(zeros_hbm, deg_sp)

    pltpu.sync_copy(row_hbm.at[c, s], ridx_v)
    for i in range(_CHUNK // 16):
        ones_v[pl.ds(i * 16, 16)] = jnp.ones((16,), jnp.float32)
    plsc.subcore_barrier()

    def body(j, carry):
        pltpu.sync_copy(ones_v, deg_sp.at[ridx_v.at[j]], add=True)
        return carry

    lax.fori_loop(0, _NCHUNK, body, 0)
    plsc.subcore_barrier()

    @pl.when(s == 0)
    def _():
        pltpu.sync_copy(deg_sp.at[pl.ds(0, _N)], out_hbm.at[c])


@functools.partial(
    pl.kernel,
    out_type=jax.ShapeDtypeStruct((_NC, _N, 16), jnp.float32),
    mesh=_mesh,
    compiler_params=_sc_params,
    scratch_types=[
        pltpu.VMEM((_NCHUNK, _CHUNK), jnp.int32),
        pltpu.VMEM((_NCHUNK, _CHUNK), jnp.int32),
        [pltpu.VMEM((_CHUNK, 16), jnp.float32) for _ in range(4)],
        pltpu.VMEM_SHARED((_NACC, 16), jnp.float32),
        [pltpu.SemaphoreType.DMA for _ in range(4)],
    ],
)
def _sc_gather_scatter(row_hbm, col_hbm, table_hbm, zeros_hbm, out_hbm,
                       ridx_v, cidx_v, bufs, acc_sp, sems):
    c = lax.axis_index("c")
    s = lax.axis_index("s")

    @pl.when(s == 0)
    def _():
        pltpu.sync_copy(zeros_hbm, acc_sp)

    pltpu.sync_copy(row_hbm.at[c, s], ridx_v)
    pltpu.sync_copy(col_hbm.at[c, s], cidx_v)
    plsc.subcore_barrier()

    def fire(j, slot):
        pltpu.async_copy(table_hbm.at[cidx_v.at[j]], bufs[slot], sems[slot])

    def wait_gather(slot):
        pltpu.make_async_copy(
            table_hbm.at[cidx_v.at[0]], bufs[slot], sems[slot]).wait()

    for k in range(3):
        fire(k, k)

    def body(i, carry):
        for k in range(4):
            j = 4 * i + k
            wait_gather(k)
            pltpu.sync_copy(bufs[k], acc_sp.at[ridx_v.at[j]], add=True)

            @pl.when(j + 3 < _NCHUNK)
            def _():
                fire(j + 3, (k + 3) % 4)
        return carry

    lax.fori_loop(0, _NCHUNK // 4, body, 0)
    wait_gather(0)
    pltpu.sync_copy(bufs[0], acc_sp.at[ridx_v.at[_NCHUNK - 1]], add=True)
    plsc.subcore_barrier()

    @pl.when(s < _NS - 1)
    def _():
        rows = pl.ds(s * _ROWS_PER_TILE, _ROWS_PER_TILE)
        pltpu.sync_copy(acc_sp.at[rows], out_hbm.at[c].at[rows])

    @pl.when(s == _NS - 1)
    def _():
        last = pl.ds((_NS - 1) * _ROWS_PER_TILE, _N - (_NS - 1) * _ROWS_PER_TILE)
        pltpu.sync_copy(acc_sp.at[last], out_hbm.at[c].at[last])


def _mm1_body(x_ref, w0_ref, w1_ref, z_ref, y_ref):
    xb = x_ref[...]
    z_ref[...] = jnp.dot(xb, w0_ref[...], preferred_element_type=jnp.float32)
    y_ref[...] = jnp.dot(xb, w1_ref[...], preferred_element_type=jnp.float32)


def _tc_mm1(x, w0, w1):
    blk = 1000
    return pl.pallas_call(
        _mm1_body,
        grid=(_N // blk,),
        in_specs=[
            pl.BlockSpec((blk, 128), lambda i: (i, 0)),
            pl.BlockSpec((128, 16), lambda i: (0, 0)),
            pl.BlockSpec((128, 16), lambda i: (0, 0)),
        ],
        out_specs=[
            pl.BlockSpec((blk, 16), lambda i: (i, 0)),
            pl.BlockSpec((blk, 16), lambda i: (i, 0)),
        ],
        out_shape=[
            jax.ShapeDtypeStruct((_N, 16), jnp.float32),
            jax.ShapeDtypeStruct((_N, 16), jnp.float32),
        ],
    )(x, w0, w1)


def _prep_body(degp_ref, y1_ref, dis_ref, y1p_ref):
    deg = degp_ref[:, 0:1] + degp_ref[:, 1:2]
    dis = jnp.where(deg > 0.0, lax.rsqrt(jnp.where(deg > 0.0, deg, 1.0)), 0.0)
    dis_ref[...] = dis
    y1p_ref[...] = dis * y1_ref[...]


def _tc_prep(degp_t, y1):
    return pl.pallas_call(
        _prep_body,
        out_shape=[
            jax.ShapeDtypeStruct((_N, 1), jnp.float32),
            jax.ShapeDtypeStruct((_N, 16), jnp.float32),
        ],
    )(degp_t, y1)


def _l1_body(z_ref, sp_ref, dis_ref, b_ref, h_ref, g_ref):
    ssum = sp_ref[0] + sp_ref[1]
    h = jnp.maximum(z_ref[...] - dis_ref[...] * ssum + b_ref[...], 0.0)
    h_ref[...] = h
    g_ref[...] = dis_ref[...] * h


def _tc_layer1(z1, s1p, dis, b1):
    return pl.pallas_call(
        _l1_body,
        out_shape=[
            jax.ShapeDtypeStruct((_N, 16), jnp.float32),
            jax.ShapeDtypeStruct((_N, 16), jnp.float32),
        ],
    )(z1, s1p, dis, b1)


def _l2_body(h_ref, sp_ref, dis_ref, w0_ref, w1_ref, b_ref, out_ref):
    t2 = -dis_ref[...] * (sp_ref[0] + sp_ref[1])
    logits = (
        jnp.dot(h_ref[...], w0_ref[...], preferred_element_type=jnp.float32)
        + jnp.dot(t2, w1_ref[...], preferred_element_type=jnp.float32)
        + b_ref[...]
    )
    m = jnp.max(logits, axis=1, keepdims=True)
    e = jnp.exp(logits - m)
    lse = jnp.log(jnp.sum(e, axis=1, keepdims=True))
    out_ref[...] = logits - m - lse


def _tc_layer2(h, s2p, dis, w0, w1, b2):
    return pl.pallas_call(
        _l2_body,
        out_shape=jax.ShapeDtypeStruct((_N, 64), jnp.float32),
    )(h, s2p, dis, w0, w1, b2)


def kernel(x, edge_index, W0_1, W1_1, b1, W0_2, W1_2, b2):
    npad = _EPAD - _E
    row = jnp.concatenate([edge_index[0], jnp.full((npad,), _N, jnp.int32)])
    col = jnp.concatenate([edge_index[1], jnp.zeros((npad,), jnp.int32)])
    row = row.reshape(_NC, _NS, _NCHUNK, _CHUNK)
    col = col.reshape(_NC, _NS, _NCHUNK, _CHUNK)
    zeros1 = jnp.zeros((_NACC,), jnp.float32)
    zeros2 = jnp.zeros((_NACC, 16), jnp.float32)

    degp = _sc_degree(row, zeros1)
    z1, y1 = _tc_mm1(x, W0_1, W1_1)
    dis, y1p = _tc_prep(degp.T, y1)
    s1p = _sc_gather_scatter(row, col, y1p, zeros2)
    h, g = _tc_layer1(z1, s1p, dis, b1)
    s2p = _sc_gather_scatter(row, col, g, zeros2)
    return _tc_layer2(h, s2p, dis, W0_2, W1_2, b2)
